# baseline (speedup 1.0000x reference)
import functools

import jax
import jax.numpy as jnp
from jax import lax
from jax.experimental import pallas as pl
from jax.experimental.pallas import tpu as pltpu
from jax.experimental.pallas import tpu_sc as plsc

N = 10000
E = 320000
F = 128
FH = F // 2

NC = 2
NS = 16
NW = NC * NS

WIN = 80
NRING = 5

EPT = E // NS
NWIN = EPT // WIN
MP_NB = 50
MP_NBLK = NWIN // MP_NB


RPT = 632
NP = NS * RPT
RPT_LAST = N - 15 * RPT

_mesh = plsc.VectorSubcoreMesh(core_axis_name="c", subcore_axis_name="s")


DEG_RPT = 640
DEG_NP = NS * DEG_RPT


def _deg_body(dst_hbm, deg_out, dst_v, ones_w, zeros_w, deg):
    c = lax.axis_index("c")
    s = lax.axis_index("s")

    @pl.loop(0, WIN)
    def _(i):
        ones_w[i, :] = jnp.ones((16,), jnp.float32)

    @pl.loop(0, DEG_RPT)
    def _(i):
        zeros_w[i, :] = jnp.zeros((16,), jnp.float32)

    pltpu.sync_copy(zeros_w, deg.at[pl.ds(s * DEG_RPT, DEG_RPT)])
    plsc.subcore_barrier()

    @pl.loop(0, MP_NBLK)
    def _(b):
        take = jnp.logical_or(b == MP_NBLK - 1, lax.rem(b, 2) == c)

        @pl.when(take)
        def _():
            pltpu.sync_copy(dst_hbm.at[s, b], dst_v)
            last = b == MP_NBLK - 1
            lo = jnp.where(last, c * (MP_NB // 2), 0)
            hi = jnp.where(last, (c + 1) * (MP_NB // 2), MP_NB)

            @pl.loop(lo, hi)
            def _(j):
                pltpu.sync_copy(ones_w, deg.at[dst_v.at[j]], add=True)

    plsc.subcore_barrier()
    pltpu.sync_copy(deg.at[pl.ds(s * DEG_RPT, DEG_RPT)],
                    deg_out.at[c, pl.ds(s * DEG_RPT, DEG_RPT)])


_deg_kernel = functools.partial(
    pl.kernel,
    out_type=jax.ShapeDtypeStruct((NC, DEG_NP, 16), jnp.float32),
    mesh=_mesh,
    scratch_types=[
        pltpu.VMEM((MP_NB, WIN), jnp.int32),
        pltpu.VMEM((WIN, 16), jnp.float32),
        pltpu.VMEM((DEG_RPT, 16), jnp.float32),
        pltpu.VMEM_SHARED((DEG_NP, 16), jnp.float32),
    ],
    compiler_params=pltpu.CompilerParams(use_tc_tiling_on_sc=False),
)(_deg_body)


def _mp_half(hp_hbm, src_hbm, dst_hbm, out_hbm, src_b, dst_b, bufs, acc, sems, s):
    r0 = s * RPT

    def start_g(w, k):
        pltpu.async_copy(hp_hbm.at[src_b.at[w]], bufs[k], sems[k])

    def wait_g(w, k):
        pltpu.make_async_copy(hp_hbm.at[src_b.at[w]], bufs[k], sems[k]).wait()

    pltpu.sync_copy(src_hbm.at[s, 0], src_b)
    pltpu.sync_copy(dst_hbm.at[s, 0], dst_b)
    for k in range(NRING):
        start_g(k, k)

    @pl.when(s < NS - 1)
    def _():
        pltpu.sync_copy(hp_hbm.at[pl.ds(r0, RPT)], acc.at[pl.ds(r0, RPT)])

    @pl.when(s == NS - 1)
    def _():
        pltpu.sync_copy(hp_hbm.at[pl.ds(r0, RPT_LAST)],
                        acc.at[pl.ds(r0, RPT_LAST)])

    plsc.subcore_barrier()

    @pl.loop(0, MP_NBLK)
    def _(b):
        @pl.when(b > 0)
        def _():
            pltpu.sync_copy(src_hbm.at[s, b], src_b)
            pltpu.sync_copy(dst_hbm.at[s, b], dst_b)

            for k in range(NRING):
                start_g(k, k)

        @pl.loop(0, MP_NB, step=NRING)
        def _(j):
            for k in range(NRING):
                w = j + k
                wait_g(w, k)
                pltpu.sync_copy(bufs[k], acc.at[dst_b.at[w]], add=True)

                @pl.when(w + NRING < MP_NB)
                def _():
                    start_g(w + NRING, k)

    plsc.subcore_barrier()

    @pl.when(s < NS - 1)
    def _():
        pltpu.sync_copy(acc.at[pl.ds(r0, RPT)], out_hbm.at[pl.ds(r0, RPT)])

    @pl.when(s == NS - 1)
    def _():
        pltpu.sync_copy(acc.at[pl.ds(r0, RPT_LAST)],
                        out_hbm.at[pl.ds(r0, RPT_LAST)])


def _mp_body(hp0_hbm, hp1_hbm, src_hbm, dst_hbm, out0_hbm, out1_hbm,
             src_b, dst_b,
             b0, b1, b2, b3, b4, acc,
             s0, s1, s2, s3, s4):
    c = lax.axis_index("c")
    s = lax.axis_index("s")
    bufs = [b0, b1, b2, b3, b4]
    sems = [s0, s1, s2, s3, s4]

    @pl.when(c == 0)
    def _():
        _mp_half(hp0_hbm, src_hbm, dst_hbm, out0_hbm, src_b, dst_b,
                 bufs, acc, sems, s)

    @pl.when(c == 1)
    def _():
        _mp_half(hp1_hbm, src_hbm, dst_hbm, out1_hbm, src_b, dst_b,
                 bufs, acc, sems, s)


_mp_kernel = functools.partial(
    pl.kernel,
    out_type=[jax.ShapeDtypeStruct((NP, FH), jnp.float32),
              jax.ShapeDtypeStruct((NP, FH), jnp.float32)],
    mesh=_mesh,
    scratch_types=(
        [
            pltpu.VMEM((MP_NB, WIN), jnp.int32),
            pltpu.VMEM((MP_NB, WIN), jnp.int32),
        ]
        + [pltpu.VMEM((WIN, FH), jnp.float32) for _ in range(NRING)]
        + [pltpu.VMEM_SHARED((NP, FH), jnp.float32)]
        + [pltpu.SemaphoreType.DMA for _ in range(NRING)]
    ),
    compiler_params=pltpu.CompilerParams(use_tc_tiling_on_sc=False),
)(_mp_body)


BR = 1000
GRID = N // BR


def _prep_body(x_ref, w_ref, deg_ref, o0_ref, o1_ref):
    dinv = lax.rsqrt(deg_ref[0, :, 0] + deg_ref[1, :, 0] + 1.0)
    h = jnp.dot(x_ref[...], w_ref[...], preferred_element_type=jnp.float32)
    hp = h * dinv[:, None]
    o0_ref[...] = hp[:, :FH]
    o1_ref[...] = hp[:, FH:]


def _mid_body(acc0_ref, acc1_ref, deg_ref, b_ref, w_ref, o0_ref, o1_ref):
    dinv = lax.rsqrt(deg_ref[0, :, 0] + deg_ref[1, :, 0] + 1.0)
    accfull = jnp.concatenate([acc0_ref[...], acc1_ref[...]], axis=1)
    z = dinv[:, None] * accfull + b_ref[...][None, :]
    g = jax.nn.gelu(z)
    h = jnp.dot(g, w_ref[...], preferred_element_type=jnp.float32)
    hp = h * dinv[:, None]
    o0_ref[...] = hp[:, :FH]
    o1_ref[...] = hp[:, FH:]


def _final_body(acc0_ref, acc1_ref, deg_ref, b_ref, o_ref):
    dinv = lax.rsqrt(deg_ref[0, :, 0] + deg_ref[1, :, 0] + 1.0)
    accfull = jnp.concatenate([acc0_ref[...], acc1_ref[...]], axis=1)
    o_ref[...] = dinv[:, None] * accfull + b_ref[...][None, :]


_row_spec = pl.BlockSpec((BR, F), lambda i: (i, 0))
_half_spec = pl.BlockSpec((BR, FH), lambda i: (i, 0))
_deg_spec = pl.BlockSpec((NC, BR, 16), lambda i: (0, i, 0))
_acc_spec = pl.BlockSpec((BR, FH), lambda i: (i, 0))
_w_spec = pl.BlockSpec((F, F), lambda i: (0, 0))
_b_spec = pl.BlockSpec((F,), lambda i: (0,))

_half_shape = jax.ShapeDtypeStruct((N, FH), jnp.float32)

_prep_kernel = pl.pallas_call(
    _prep_body,
    grid=(GRID,),
    in_specs=[_row_spec, _w_spec, _deg_spec],
    out_specs=[_half_spec, _half_spec],
    out_shape=[_half_shape, _half_shape],
)

_mid_kernel = pl.pallas_call(
    _mid_body,
    grid=(GRID,),
    in_specs=[_acc_spec, _acc_spec, _deg_spec, _b_spec, _w_spec],
    out_specs=[_half_spec, _half_spec],
    out_shape=[_half_shape, _half_shape],
)

_final_kernel = pl.pallas_call(
    _final_body,
    grid=(GRID,),
    in_specs=[_acc_spec, _acc_spec, _deg_spec, _b_spec],
    out_specs=_row_spec,
    out_shape=jax.ShapeDtypeStruct((N, F), jnp.float32),
)


@jax.jit
def kernel(x, edge_index, W1, b1, W2, b2):
    src_mp = edge_index[0].reshape(NS, MP_NBLK, MP_NB, WIN)
    dst_mp = edge_index[1].reshape(NS, MP_NBLK, MP_NB, WIN)

    deg2 = _deg_kernel(dst_mp)
    h1p0, h1p1 = _prep_kernel(x, W1, deg2)
    a10, a11 = _mp_kernel(h1p0, h1p1, src_mp, dst_mp)
    h2p0, h2p1 = _mid_kernel(a10, a11, deg2, b1, W2)
    a20, a21 = _mp_kernel(h2p0, h2p1, src_mp, dst_mp)
    return _final_kernel(a20, a21, deg2, b2)


# speedup vs baseline: 1.0243x; 1.0243x over previous
import functools

import jax
import jax.numpy as jnp
from jax import lax
from jax.experimental import pallas as pl
from jax.experimental.pallas import tpu as pltpu
from jax.experimental.pallas import tpu_sc as plsc

N = 10000
E = 320000
F = 128
FH = F // 2

NC = 2
NS = 16
NW = NC * NS

WIN = 80
NRING = 5

EPT = E // NS
NWIN = EPT // WIN
MP_NB = 50
MP_NBLK = NWIN // MP_NB


RPT = 632
NP = NS * RPT
RPT_LAST = N - 15 * RPT

_mesh = plsc.VectorSubcoreMesh(core_axis_name="c", subcore_axis_name="s")


DEG_RPT = 640
DEG_NP = NS * DEG_RPT


def _deg_body(dst_hbm, deg_out, dst_v, ones_w, zeros_w, deg):
    c = lax.axis_index("c")
    s = lax.axis_index("s")

    @pl.loop(0, WIN // 16)
    def _(i):
        ones_w[pl.ds(i * 16, 16)] = jnp.ones((16,), jnp.float32)

    @pl.loop(0, DEG_RPT // 16)
    def _(i):
        zeros_w[pl.ds(i * 16, 16)] = jnp.zeros((16,), jnp.float32)

    pltpu.sync_copy(zeros_w, deg.at[pl.ds(s * DEG_RPT, DEG_RPT)])
    plsc.subcore_barrier()

    @pl.loop(0, MP_NBLK)
    def _(b):
        take = jnp.logical_or(b == MP_NBLK - 1, lax.rem(b, 2) == c)

        @pl.when(take)
        def _():
            pltpu.sync_copy(dst_hbm.at[s, b], dst_v)
            last = b == MP_NBLK - 1
            lo = jnp.where(last, c * (MP_NB // 2), 0)
            hi = jnp.where(last, (c + 1) * (MP_NB // 2), MP_NB)

            @pl.loop(lo, hi)
            def _(j):
                pltpu.sync_copy(ones_w, deg.at[dst_v.at[j]], add=True)

    plsc.subcore_barrier()
    pltpu.sync_copy(deg.at[pl.ds(s * DEG_RPT, DEG_RPT)],
                    deg_out.at[c, pl.ds(s * DEG_RPT, DEG_RPT)])


_deg_kernel = functools.partial(
    pl.kernel,
    out_type=jax.ShapeDtypeStruct((NC, DEG_NP), jnp.float32),
    mesh=_mesh,
    scratch_types=[
        pltpu.VMEM((MP_NB, WIN), jnp.int32),
        pltpu.VMEM((WIN,), jnp.float32),
        pltpu.VMEM((DEG_RPT,), jnp.float32),
        pltpu.VMEM_SHARED((DEG_NP,), jnp.float32),
    ],
    compiler_params=pltpu.CompilerParams(use_tc_tiling_on_sc=False),
)(_deg_body)


def _mp_half(hp_hbm, src_hbm, dst_hbm, out_hbm, src_b, dst_b, bufs, acc, sems, s):
    r0 = s * RPT

    def start_g(w, k):
        pltpu.async_copy(hp_hbm.at[src_b.at[w]], bufs[k], sems[k])

    def wait_g(w, k):
        pltpu.make_async_copy(hp_hbm.at[src_b.at[w]], bufs[k], sems[k]).wait()

    pltpu.sync_copy(src_hbm.at[s, 0], src_b)
    pltpu.sync_copy(dst_hbm.at[s, 0], dst_b)
    for k in range(NRING):
        start_g(k, k)

    @pl.when(s < NS - 1)
    def _():
        pltpu.sync_copy(hp_hbm.at[pl.ds(r0, RPT)], acc.at[pl.ds(r0, RPT)])

    @pl.when(s == NS - 1)
    def _():
        pltpu.sync_copy(hp_hbm.at[pl.ds(r0, RPT_LAST)],
                        acc.at[pl.ds(r0, RPT_LAST)])

    plsc.subcore_barrier()

    @pl.loop(0, MP_NBLK)
    def _(b):
        @pl.when(b > 0)
        def _():
            pltpu.sync_copy(src_hbm.at[s, b], src_b)
            pltpu.sync_copy(dst_hbm.at[s, b], dst_b)

            for k in range(NRING):
                start_g(k, k)

        @pl.loop(0, MP_NB, step=NRING)
        def _(j):
            for k in range(NRING):
                w = j + k
                wait_g(w, k)
                pltpu.sync_copy(bufs[k], acc.at[dst_b.at[w]], add=True)

                @pl.when(w + NRING < MP_NB)
                def _():
                    start_g(w + NRING, k)

    plsc.subcore_barrier()

    @pl.when(s < NS - 1)
    def _():
        pltpu.sync_copy(acc.at[pl.ds(r0, RPT)], out_hbm.at[pl.ds(r0, RPT)])

    @pl.when(s == NS - 1)
    def _():
        pltpu.sync_copy(acc.at[pl.ds(r0, RPT_LAST)],
                        out_hbm.at[pl.ds(r0, RPT_LAST)])


def _mp_body(hp0_hbm, hp1_hbm, src_hbm, dst_hbm, out0_hbm, out1_hbm,
             src_b, dst_b,
             b0, b1, b2, b3, b4, acc,
             s0, s1, s2, s3, s4):
    c = lax.axis_index("c")
    s = lax.axis_index("s")
    bufs = [b0, b1, b2, b3, b4]
    sems = [s0, s1, s2, s3, s4]

    @pl.when(c == 0)
    def _():
        _mp_half(hp0_hbm, src_hbm, dst_hbm, out0_hbm, src_b, dst_b,
                 bufs, acc, sems, s)

    @pl.when(c == 1)
    def _():
        _mp_half(hp1_hbm, src_hbm, dst_hbm, out1_hbm, src_b, dst_b,
                 bufs, acc, sems, s)


_mp_kernel = functools.partial(
    pl.kernel,
    out_type=[jax.ShapeDtypeStruct((NP, FH), jnp.float32),
              jax.ShapeDtypeStruct((NP, FH), jnp.float32)],
    mesh=_mesh,
    scratch_types=(
        [
            pltpu.VMEM((MP_NB, WIN), jnp.int32),
            pltpu.VMEM((MP_NB, WIN), jnp.int32),
        ]
        + [pltpu.VMEM((WIN, FH), jnp.float32) for _ in range(NRING)]
        + [pltpu.VMEM_SHARED((NP, FH), jnp.float32)]
        + [pltpu.SemaphoreType.DMA for _ in range(NRING)]
    ),
    compiler_params=pltpu.CompilerParams(use_tc_tiling_on_sc=False),
)(_mp_body)


BR = 1000
GRID = N // BR


def _prep_body(x_ref, w_ref, deg_ref, o0_ref, o1_ref):
    dinv = lax.rsqrt(deg_ref[0, :, 0] + deg_ref[1, :, 0] + 1.0)
    h = jnp.dot(x_ref[...], w_ref[...], preferred_element_type=jnp.float32)
    hp = h * dinv[:, None]
    o0_ref[...] = hp[:, :FH]
    o1_ref[...] = hp[:, FH:]


def _mid_body(acc0_ref, acc1_ref, deg_ref, b_ref, w_ref, o0_ref, o1_ref):
    dinv = lax.rsqrt(deg_ref[0, :, 0] + deg_ref[1, :, 0] + 1.0)
    accfull = jnp.concatenate([acc0_ref[...], acc1_ref[...]], axis=1)
    z = dinv[:, None] * accfull + b_ref[...][None, :]
    g = jax.nn.gelu(z)
    h = jnp.dot(g, w_ref[...], preferred_element_type=jnp.float32)
    hp = h * dinv[:, None]
    o0_ref[...] = hp[:, :FH]
    o1_ref[...] = hp[:, FH:]


def _final_body(acc0_ref, acc1_ref, deg_ref, b_ref, o_ref):
    dinv = lax.rsqrt(deg_ref[0, :, 0] + deg_ref[1, :, 0] + 1.0)
    accfull = jnp.concatenate([acc0_ref[...], acc1_ref[...]], axis=1)
    o_ref[...] = dinv[:, None] * accfull + b_ref[...][None, :]


_row_spec = pl.BlockSpec((BR, F), lambda i: (i, 0))
_half_spec = pl.BlockSpec((BR, FH), lambda i: (i, 0))
_deg_spec = pl.BlockSpec((NC, BR, 1), lambda i: (0, i, 0))
_acc_spec = pl.BlockSpec((BR, FH), lambda i: (i, 0))
_w_spec = pl.BlockSpec((F, F), lambda i: (0, 0))
_b_spec = pl.BlockSpec((F,), lambda i: (0,))

_half_shape = jax.ShapeDtypeStruct((N, FH), jnp.float32)

_prep_kernel = pl.pallas_call(
    _prep_body,
    grid=(GRID,),
    in_specs=[_row_spec, _w_spec, _deg_spec],
    out_specs=[_half_spec, _half_spec],
    out_shape=[_half_shape, _half_shape],
)

_mid_kernel = pl.pallas_call(
    _mid_body,
    grid=(GRID,),
    in_specs=[_acc_spec, _acc_spec, _deg_spec, _b_spec, _w_spec],
    out_specs=[_half_spec, _half_spec],
    out_shape=[_half_shape, _half_shape],
)

_final_kernel = pl.pallas_call(
    _final_body,
    grid=(GRID,),
    in_specs=[_acc_spec, _acc_spec, _deg_spec, _b_spec],
    out_specs=_row_spec,
    out_shape=jax.ShapeDtypeStruct((N, F), jnp.float32),
)


@jax.jit
def kernel(x, edge_index, W1, b1, W2, b2):
    src_mp = edge_index[0].reshape(NS, MP_NBLK, MP_NB, WIN)
    dst_mp = edge_index[1].reshape(NS, MP_NBLK, MP_NB, WIN)

    deg2 = _deg_kernel(dst_mp)[:, :N].reshape(NC, N, 1)
    h1p0, h1p1 = _prep_kernel(x, W1, deg2)
    a10, a11 = _mp_kernel(h1p0, h1p1, src_mp, dst_mp)
    h2p0, h2p1 = _mid_kernel(a10, a11, deg2, b1, W2)
    a20, a21 = _mp_kernel(h2p0, h2p1, src_mp, dst_mp)
    return _final_kernel(a20, a21, deg2, b2)


# speedup vs baseline: 1.0409x; 1.0163x over previous
import functools

import jax
import jax.numpy as jnp
from jax import lax
from jax.experimental import pallas as pl
from jax.experimental.pallas import tpu as pltpu
from jax.experimental.pallas import tpu_sc as plsc

N = 10000
E = 320000
F = 128
FH = F // 2

NC = 2
NS = 16
NW = NC * NS

WIN = 80
NRING = 5

EPT = E // NS
NWIN = EPT // WIN
MP_NB = 50
MP_NBLK = NWIN // MP_NB


RPT = 632
NP = NS * RPT
RPT_LAST = N - 15 * RPT

_mesh = plsc.VectorSubcoreMesh(core_axis_name="c", subcore_axis_name="s")


DEG_RPT = 640
DEG_NP = NS * DEG_RPT


def _deg_body(dst_hbm, deg_out, dst_v, ones_w, zeros_w, deg):
    c = lax.axis_index("c")
    s = lax.axis_index("s")

    @pl.loop(0, WIN // 16)
    def _(i):
        ones_w[pl.ds(i * 16, 16)] = jnp.ones((16,), jnp.float32)

    @pl.loop(0, DEG_RPT // 16)
    def _(i):
        zeros_w[pl.ds(i * 16, 16)] = jnp.zeros((16,), jnp.float32)

    pltpu.sync_copy(zeros_w, deg.at[pl.ds(s * DEG_RPT, DEG_RPT)])
    plsc.subcore_barrier()

    @pl.loop(0, MP_NBLK)
    def _(b):
        take = jnp.logical_or(b == MP_NBLK - 1, lax.rem(b, 2) == c)

        @pl.when(take)
        def _():
            pltpu.sync_copy(dst_hbm.at[s, b], dst_v)
            last = b == MP_NBLK - 1
            lo = jnp.where(last, c * (MP_NB // 2), 0)
            hi = jnp.where(last, (c + 1) * (MP_NB // 2), MP_NB)

            @pl.loop(lo, hi)
            def _(j):
                pltpu.sync_copy(ones_w, deg.at[dst_v.at[j]], add=True)

    plsc.subcore_barrier()
    pltpu.sync_copy(deg.at[pl.ds(s * DEG_RPT, DEG_RPT)],
                    deg_out.at[c, pl.ds(s * DEG_RPT, DEG_RPT)])


_deg_kernel = functools.partial(
    pl.kernel,
    out_type=jax.ShapeDtypeStruct((NC, DEG_NP), jnp.float32),
    mesh=_mesh,
    scratch_types=[
        pltpu.VMEM((MP_NB, WIN), jnp.int32),
        pltpu.VMEM((WIN,), jnp.float32),
        pltpu.VMEM((DEG_RPT,), jnp.float32),
        pltpu.VMEM_SHARED((DEG_NP,), jnp.float32),
    ],
    compiler_params=pltpu.CompilerParams(use_tc_tiling_on_sc=False),
)(_deg_body)


def _mp_half(hp_hbm, src_hbm, dst_hbm, out_hbm, src_b, dst_b, bufs, acc, sems, s):
    r0 = s * RPT

    def start_g(w, k):
        pltpu.async_copy(hp_hbm.at[src_b.at[w]], bufs[k], sems[k])

    def wait_g(w, k):
        pltpu.make_async_copy(hp_hbm.at[src_b.at[w]], bufs[k], sems[k]).wait()

    pltpu.sync_copy(src_hbm.at[s, 0], src_b)
    pltpu.sync_copy(dst_hbm.at[s, 0], dst_b)
    for k in range(NRING):
        start_g(k, k)

    @pl.when(s < NS - 1)
    def _():
        pltpu.sync_copy(hp_hbm.at[pl.ds(r0, RPT)], acc.at[pl.ds(r0, RPT)])

    @pl.when(s == NS - 1)
    def _():
        pltpu.sync_copy(hp_hbm.at[pl.ds(r0, RPT_LAST)],
                        acc.at[pl.ds(r0, RPT_LAST)])

    plsc.subcore_barrier()

    @pl.loop(0, MP_NBLK)
    def _(b):
        @pl.when(b > 0)
        def _():
            pltpu.sync_copy(src_hbm.at[s, b], src_b)
            pltpu.sync_copy(dst_hbm.at[s, b], dst_b)

            for k in range(NRING):
                start_g(k, k)

        @pl.loop(0, MP_NB, step=NRING)
        def _(j):
            for k in range(NRING):
                w = j + k
                wait_g(w, k)
                pltpu.sync_copy(bufs[k], acc.at[dst_b.at[w]], add=True)

                @pl.when(w + NRING < MP_NB)
                def _():
                    start_g(w + NRING, k)

    plsc.subcore_barrier()

    @pl.when(s < NS - 1)
    def _():
        pltpu.sync_copy(acc.at[pl.ds(r0, RPT)], out_hbm.at[pl.ds(r0, RPT)])

    @pl.when(s == NS - 1)
    def _():
        pltpu.sync_copy(acc.at[pl.ds(r0, RPT_LAST)],
                        out_hbm.at[pl.ds(r0, RPT_LAST)])


def _mp_body(hp0_hbm, hp1_hbm, src_hbm, dst_hbm, out0_hbm, out1_hbm,
             src_b, dst_b,
             b0, b1, b2, b3, b4, acc,
             s0, s1, s2, s3, s4):
    c = lax.axis_index("c")
    s = lax.axis_index("s")
    bufs = [b0, b1, b2, b3, b4]
    sems = [s0, s1, s2, s3, s4]

    @pl.when(c == 0)
    def _():
        _mp_half(hp0_hbm, src_hbm, dst_hbm, out0_hbm, src_b, dst_b,
                 bufs, acc, sems, s)

    @pl.when(c == 1)
    def _():
        _mp_half(hp1_hbm, src_hbm, dst_hbm, out1_hbm, src_b, dst_b,
                 bufs, acc, sems, s)


_mp_kernel = functools.partial(
    pl.kernel,
    out_type=[jax.ShapeDtypeStruct((NP, FH), jnp.float32),
              jax.ShapeDtypeStruct((NP, FH), jnp.float32)],
    mesh=_mesh,
    scratch_types=(
        [
            pltpu.VMEM((MP_NB, WIN), jnp.int32),
            pltpu.VMEM((MP_NB, WIN), jnp.int32),
        ]
        + [pltpu.VMEM((WIN, FH), jnp.float32) for _ in range(NRING)]
        + [pltpu.VMEM_SHARED((NP, FH), jnp.float32)]
        + [pltpu.SemaphoreType.DMA for _ in range(NRING)]
    ),
    compiler_params=pltpu.CompilerParams(use_tc_tiling_on_sc=False),
)(_mp_body)


BR = 2000
GRID = N // BR


def _prep_body(x_ref, w_ref, deg_ref, o0_ref, o1_ref):
    dinv = lax.rsqrt(deg_ref[0, :, 0] + deg_ref[1, :, 0] + 1.0)
    h = jnp.dot(x_ref[...], w_ref[...], preferred_element_type=jnp.float32)
    hp = h * dinv[:, None]
    o0_ref[...] = hp[:, :FH]
    o1_ref[...] = hp[:, FH:]


def _mid_body(acc0_ref, acc1_ref, deg_ref, b_ref, w_ref, o0_ref, o1_ref):
    dinv = lax.rsqrt(deg_ref[0, :, 0] + deg_ref[1, :, 0] + 1.0)
    accfull = jnp.concatenate([acc0_ref[...], acc1_ref[...]], axis=1)
    z = dinv[:, None] * accfull + b_ref[...][None, :]
    g = jax.nn.gelu(z)
    h = jnp.dot(g, w_ref[...], preferred_element_type=jnp.float32)
    hp = h * dinv[:, None]
    o0_ref[...] = hp[:, :FH]
    o1_ref[...] = hp[:, FH:]


def _final_body(acc0_ref, acc1_ref, deg_ref, b_ref, o_ref):
    dinv = lax.rsqrt(deg_ref[0, :, 0] + deg_ref[1, :, 0] + 1.0)
    accfull = jnp.concatenate([acc0_ref[...], acc1_ref[...]], axis=1)
    o_ref[...] = dinv[:, None] * accfull + b_ref[...][None, :]


_row_spec = pl.BlockSpec((BR, F), lambda i: (i, 0))
_half_spec = pl.BlockSpec((BR, FH), lambda i: (i, 0))
_deg_spec = pl.BlockSpec((NC, BR, 1), lambda i: (0, i, 0))
_acc_spec = pl.BlockSpec((BR, FH), lambda i: (i, 0))
_w_spec = pl.BlockSpec((F, F), lambda i: (0, 0))
_b_spec = pl.BlockSpec((F,), lambda i: (0,))

_half_shape = jax.ShapeDtypeStruct((N, FH), jnp.float32)

_prep_kernel = pl.pallas_call(
    _prep_body,
    grid=(GRID,),
    in_specs=[_row_spec, _w_spec, _deg_spec],
    out_specs=[_half_spec, _half_spec],
    out_shape=[_half_shape, _half_shape],
)

_mid_kernel = pl.pallas_call(
    _mid_body,
    grid=(GRID,),
    in_specs=[_acc_spec, _acc_spec, _deg_spec, _b_spec, _w_spec],
    out_specs=[_half_spec, _half_spec],
    out_shape=[_half_shape, _half_shape],
)

_final_kernel = pl.pallas_call(
    _final_body,
    grid=(GRID,),
    in_specs=[_acc_spec, _acc_spec, _deg_spec, _b_spec],
    out_specs=_row_spec,
    out_shape=jax.ShapeDtypeStruct((N, F), jnp.float32),
)


@jax.jit
def kernel(x, edge_index, W1, b1, W2, b2):
    src_mp = edge_index[0].reshape(NS, MP_NBLK, MP_NB, WIN)
    dst_mp = edge_index[1].reshape(NS, MP_NBLK, MP_NB, WIN)

    deg2 = _deg_kernel(dst_mp)[:, :N].reshape(NC, N, 1)
    h1p0, h1p1 = _prep_kernel(x, W1, deg2)
    a10, a11 = _mp_kernel(h1p0, h1p1, src_mp, dst_mp)
    h2p0, h2p1 = _mid_kernel(a10, a11, deg2, b1, W2)
    a20, a21 = _mp_kernel(h2p0, h2p1, src_mp, dst_mp)
    return _final_kernel(a20, a21, deg2, b2)


# speedup vs baseline: 1.0693x; 1.0273x over previous
import functools

import jax
import jax.numpy as jnp
from jax import lax
from jax.experimental import pallas as pl
from jax.experimental.pallas import tpu as pltpu
from jax.experimental.pallas import tpu_sc as plsc

N = 10000
E = 320000
F = 128
FH = F // 2

NC = 2
NS = 16
NW = NC * NS

WIN = 80
NRING = 5

EPT = E // NS
NWIN = EPT // WIN
MP_NB = 50
MP_NBLK = NWIN // MP_NB


RPT = 632
NP = NS * RPT
RPT_LAST = N - 15 * RPT

_mesh = plsc.VectorSubcoreMesh(core_axis_name="c", subcore_axis_name="s")


DEG_RPT = 640
DEG_NP = NS * DEG_RPT


def _deg_body(dst_hbm, deg_out, dst_v, ones_w, zeros_w, deg, dsem):
    c = lax.axis_index("c")
    s = lax.axis_index("s")

    @pl.loop(0, WIN // 16)
    def _(i):
        ones_w[pl.ds(i * 16, 16)] = jnp.ones((16,), jnp.float32)

    @pl.loop(0, DEG_RPT // 16)
    def _(i):
        zeros_w[pl.ds(i * 16, 16)] = jnp.zeros((16,), jnp.float32)

    pltpu.sync_copy(zeros_w, deg.at[pl.ds(s * DEG_RPT, DEG_RPT)])
    plsc.subcore_barrier()

    @pl.loop(0, MP_NBLK)
    def _(b):
        take = jnp.logical_or(b == MP_NBLK - 1, lax.rem(b, 2) == c)

        @pl.when(take)
        def _():
            pltpu.sync_copy(dst_hbm.at[s, b], dst_v)
            last = b == MP_NBLK - 1
            lo = jnp.where(last, c * (MP_NB // 2), 0)
            hi = jnp.where(last, (c + 1) * (MP_NB // 2), MP_NB)

            @pl.loop(lo, hi)
            def _(j):
                pltpu.async_copy(ones_w, deg.at[dst_v.at[j]], dsem, add=True)

            @pl.when(jnp.logical_not(last))
            def _():
                pltpu.make_async_copy(dst_hbm.at[s, b], dst_v, dsem).wait()

            @pl.when(last)
            def _():
                pltpu.make_async_copy(
                    dst_hbm.at[s, b, pl.ds(0, MP_NB // 2)],
                    dst_v.at[pl.ds(0, MP_NB // 2)], dsem).wait()

    plsc.subcore_barrier()
    pltpu.sync_copy(deg.at[pl.ds(s * DEG_RPT, DEG_RPT)],
                    deg_out.at[c, pl.ds(s * DEG_RPT, DEG_RPT)])


_deg_kernel = functools.partial(
    pl.kernel,
    out_type=jax.ShapeDtypeStruct((NC, DEG_NP), jnp.float32),
    mesh=_mesh,
    scratch_types=[
        pltpu.VMEM((MP_NB, WIN), jnp.int32),
        pltpu.VMEM((WIN,), jnp.float32),
        pltpu.VMEM((DEG_RPT,), jnp.float32),
        pltpu.VMEM_SHARED((DEG_NP,), jnp.float32),
        pltpu.SemaphoreType.DMA,
    ],
    compiler_params=pltpu.CompilerParams(use_tc_tiling_on_sc=False),
)(_deg_body)


def _mp_half(hp_hbm, src_hbm, dst_hbm, out_hbm, src_b, dst_b, bufs, acc, sems, s):
    r0 = s * RPT

    def start_g(w, k):
        pltpu.async_copy(hp_hbm.at[src_b.at[w]], bufs[k], sems[k])

    def wait_g(w, k):
        pltpu.make_async_copy(hp_hbm.at[src_b.at[w]], bufs[k], sems[k]).wait()

    pltpu.sync_copy(src_hbm.at[s, 0], src_b)
    pltpu.sync_copy(dst_hbm.at[s, 0], dst_b)
    for k in range(NRING):
        start_g(k, k)

    @pl.when(s < NS - 1)
    def _():
        pltpu.sync_copy(hp_hbm.at[pl.ds(r0, RPT)], acc.at[pl.ds(r0, RPT)])

    @pl.when(s == NS - 1)
    def _():
        pltpu.sync_copy(hp_hbm.at[pl.ds(r0, RPT_LAST)],
                        acc.at[pl.ds(r0, RPT_LAST)])

    plsc.subcore_barrier()

    @pl.loop(0, MP_NBLK)
    def _(b):
        @pl.when(b > 0)
        def _():
            pltpu.sync_copy(src_hbm.at[s, b], src_b)
            pltpu.sync_copy(dst_hbm.at[s, b], dst_b)

            for k in range(NRING):
                start_g(k, k)

        @pl.loop(0, MP_NB, step=NRING)
        def _(j):
            for k in range(NRING):
                w = j + k
                wait_g(w, k)
                pltpu.sync_copy(bufs[k], acc.at[dst_b.at[w]], add=True)

                @pl.when(w + NRING < MP_NB)
                def _():
                    start_g(w + NRING, k)

    plsc.subcore_barrier()

    @pl.when(s < NS - 1)
    def _():
        pltpu.sync_copy(acc.at[pl.ds(r0, RPT)], out_hbm.at[pl.ds(r0, RPT)])

    @pl.when(s == NS - 1)
    def _():
        pltpu.sync_copy(acc.at[pl.ds(r0, RPT_LAST)],
                        out_hbm.at[pl.ds(r0, RPT_LAST)])


def _mp_body(hp0_hbm, hp1_hbm, src_hbm, dst_hbm, out0_hbm, out1_hbm,
             src_b, dst_b,
             b0, b1, b2, b3, b4, acc,
             s0, s1, s2, s3, s4):
    c = lax.axis_index("c")
    s = lax.axis_index("s")
    bufs = [b0, b1, b2, b3, b4]
    sems = [s0, s1, s2, s3, s4]

    @pl.when(c == 0)
    def _():
        _mp_half(hp0_hbm, src_hbm, dst_hbm, out0_hbm, src_b, dst_b,
                 bufs, acc, sems, s)

    @pl.when(c == 1)
    def _():
        _mp_half(hp1_hbm, src_hbm, dst_hbm, out1_hbm, src_b, dst_b,
                 bufs, acc, sems, s)


_mp_kernel = functools.partial(
    pl.kernel,
    out_type=[jax.ShapeDtypeStruct((NP, FH), jnp.float32),
              jax.ShapeDtypeStruct((NP, FH), jnp.float32)],
    mesh=_mesh,
    scratch_types=(
        [
            pltpu.VMEM((MP_NB, WIN), jnp.int32),
            pltpu.VMEM((MP_NB, WIN), jnp.int32),
        ]
        + [pltpu.VMEM((WIN, FH), jnp.float32) for _ in range(NRING)]
        + [pltpu.VMEM_SHARED((NP, FH), jnp.float32)]
        + [pltpu.SemaphoreType.DMA for _ in range(NRING)]
    ),
    compiler_params=pltpu.CompilerParams(use_tc_tiling_on_sc=False),
)(_mp_body)


BR = 2000
GRID = N // BR


def _prep_body(x_ref, w_ref, deg_ref, o0_ref, o1_ref):
    dinv = lax.rsqrt(deg_ref[0, :, 0] + deg_ref[1, :, 0] + 1.0)
    h = jnp.dot(x_ref[...], w_ref[...], preferred_element_type=jnp.float32)
    hp = h * dinv[:, None]
    o0_ref[...] = hp[:, :FH]
    o1_ref[...] = hp[:, FH:]


def _mid_body(acc0_ref, acc1_ref, deg_ref, b_ref, w_ref, o0_ref, o1_ref):
    dinv = lax.rsqrt(deg_ref[0, :, 0] + deg_ref[1, :, 0] + 1.0)
    accfull = jnp.concatenate([acc0_ref[...], acc1_ref[...]], axis=1)
    z = dinv[:, None] * accfull + b_ref[...][None, :]
    g = jax.nn.gelu(z)
    h = jnp.dot(g, w_ref[...], preferred_element_type=jnp.float32)
    hp = h * dinv[:, None]
    o0_ref[...] = hp[:, :FH]
    o1_ref[...] = hp[:, FH:]


def _final_body(acc0_ref, acc1_ref, deg_ref, b_ref, o_ref):
    dinv = lax.rsqrt(deg_ref[0, :, 0] + deg_ref[1, :, 0] + 1.0)
    accfull = jnp.concatenate([acc0_ref[...], acc1_ref[...]], axis=1)
    o_ref[...] = dinv[:, None] * accfull + b_ref[...][None, :]


_row_spec = pl.BlockSpec((BR, F), lambda i: (i, 0))
_half_spec = pl.BlockSpec((BR, FH), lambda i: (i, 0))
_deg_spec = pl.BlockSpec((NC, BR, 1), lambda i: (0, i, 0))
_acc_spec = pl.BlockSpec((BR, FH), lambda i: (i, 0))
_w_spec = pl.BlockSpec((F, F), lambda i: (0, 0))
_b_spec = pl.BlockSpec((F,), lambda i: (0,))

_half_shape = jax.ShapeDtypeStruct((N, FH), jnp.float32)

_prep_kernel = pl.pallas_call(
    _prep_body,
    grid=(GRID,),
    in_specs=[_row_spec, _w_spec, _deg_spec],
    out_specs=[_half_spec, _half_spec],
    out_shape=[_half_shape, _half_shape],
)

_mid_kernel = pl.pallas_call(
    _mid_body,
    grid=(GRID,),
    in_specs=[_acc_spec, _acc_spec, _deg_spec, _b_spec, _w_spec],
    out_specs=[_half_spec, _half_spec],
    out_shape=[_half_shape, _half_shape],
)

_final_kernel = pl.pallas_call(
    _final_body,
    grid=(GRID,),
    in_specs=[_acc_spec, _acc_spec, _deg_spec, _b_spec],
    out_specs=_row_spec,
    out_shape=jax.ShapeDtypeStruct((N, F), jnp.float32),
)


@jax.jit
def kernel(x, edge_index, W1, b1, W2, b2):
    src_mp = edge_index[0].reshape(NS, MP_NBLK, MP_NB, WIN)
    dst_mp = edge_index[1].reshape(NS, MP_NBLK, MP_NB, WIN)

    deg2 = _deg_kernel(dst_mp)[:, :N].reshape(NC, N, 1)
    h1p0, h1p1 = _prep_kernel(x, W1, deg2)
    a10, a11 = _mp_kernel(h1p0, h1p1, src_mp, dst_mp)
    h2p0, h2p1 = _mid_kernel(a10, a11, deg2, b1, W2)
    a20, a21 = _mp_kernel(h2p0, h2p1, src_mp, dst_mp)
    return _final_kernel(a20, a21, deg2, b2)
